# baseline (speedup 1.0000x reference)
import jax
import jax.numpy as jnp
from jax import lax
from jax.experimental import pallas as pl
from jax.experimental.pallas import tpu as pltpu
from jax.experimental.pallas import tpu_sc as plsc

N = 21
B, H, W = 8, 512, 512
BH = 256
NBINS = N * N
PADB = 512
NC, NS = 2, 16
NW = NC * NS
VLEN = 16
TOTAL = B * H * W
CHUNK = TOTAL // NW


def _argmax_body(p_ref, g_ref, o_ref):
    x = p_ref[0]
    m = jnp.max(x, axis=0)
    ch = lax.broadcasted_iota(jnp.int32, (N, BH, W), 0)
    am = jnp.min(jnp.where(x == m[None], ch, N), axis=0)
    o_ref[0] = g_ref[0] * N + am


NCH = 4


def _hist_body(idx_hbm, out_hbm, idx_v, hist_v, acc_v, *sems):
    nb = idx_hbm.shape[0]
    slabs = NW // nb
    rows = H // slabs
    crows = rows // NCH
    wid = lax.axis_index("c") * NS + lax.axis_index("s")

    b = wid // slabs
    r0 = (wid % slabs) * rows
    copies = [
        pltpu.async_copy(
            idx_hbm.at[b, pl.ds(r0 + k * crows, crows), :],
            idx_v.at[pl.ds(k * crows, crows), :], sems[k])
        for k in range(NCH)
    ]

    @plsc.parallel_loop(0, VLEN * PADB, step=VLEN, unroll=8)
    def _zero(i):
        hist_v[pl.ds(i, VLEN)] = jnp.zeros((VLEN,), jnp.float32)

    lanebase = lax.iota(jnp.int32, VLEN) * PADB
    ones = jnp.ones((VLEN,), jnp.float32)

    for k in range(NCH):
        copies[k].wait()

        @plsc.parallel_loop(k * crows, (k + 1) * crows, step=1, unroll=4)
        def _scat(i):
            for c in range(W // VLEN):
                v = idx_v[i, pl.ds(c * VLEN, VLEN)]
                plsc.addupdate_scatter(hist_v, [lanebase + v], ones)

    @plsc.parallel_loop(0, PADB, step=VLEN, unroll=2)
    def _red(cc):
        acc = hist_v[pl.ds(cc, VLEN)]
        for l in range(1, VLEN):
            acc = acc + hist_v[pl.ds(l * PADB + cc, VLEN)]
        acc_v[pl.ds(cc, VLEN)] = acc

    pltpu.sync_copy(acc_v, out_hbm.at[wid])


def _iou_from_conf(conf, o_ref):
    confb = jnp.broadcast_to(conf, (N, PADB))
    b = lax.broadcasted_iota(jnp.int32, (N, PADB), 1)
    c = lax.broadcasted_iota(jnp.int32, (N, PADB), 0)
    valid = b < NBINS
    rowm = ((b // N) == c) & valid
    colm = (lax.rem(b, N) == c) & valid
    tpm = rowm & colm
    z = jnp.zeros((N, PADB), jnp.float32)
    rowsum = jnp.sum(jnp.where(rowm, confb, z), axis=1, keepdims=True)
    colsum = jnp.sum(jnp.where(colm, confb, z), axis=1, keepdims=True)
    tp = jnp.sum(jnp.where(tpm, confb, z), axis=1, keepdims=True)
    denom = rowsum + colsum - tp
    present = denom > 0.0
    iou = jnp.where(present, tp / jnp.where(present, denom, 1.0), 0.0)
    miou = jnp.sum(iou) / jnp.sum(present.astype(jnp.float32))
    o_ref[0, 0] = miou


def _stage1(preds, gts, b0, nb):
    return pl.pallas_call(
        _argmax_body,
        grid=(nb, H // BH),
        in_specs=[
            pl.BlockSpec((1, N, BH, W), lambda b, h: (b + b0, 0, h, 0)),
            pl.BlockSpec((1, BH, W), lambda b, h: (b + b0, h, 0)),
        ],
        out_specs=pl.BlockSpec((1, BH, W), lambda b, h: (b, h, 0)),
        out_shape=jax.ShapeDtypeStruct((nb, H, W), jnp.int32),
    )(preds, gts)


def _stage2(idx):
    nb = idx.shape[0]
    rows = H // (NW // nb)
    mesh = plsc.VectorSubcoreMesh(core_axis_name="c", subcore_axis_name="s")
    return pl.kernel(
        _hist_body,
        out_type=jax.ShapeDtypeStruct((NW, PADB), jnp.float32),
        mesh=mesh,
        scratch_types=[
            pltpu.VMEM((rows, W), jnp.int32),
            pltpu.VMEM((VLEN * PADB,), jnp.float32),
            pltpu.VMEM((PADB,), jnp.float32),
        ] + [pltpu.SemaphoreType.DMA] * NCH,
        compiler_params=pltpu.CompilerParams(
            needs_layout_passes=False, use_tc_tiling_on_sc=True),
    )(idx)


def _iou_multi_body(*refs):
    o_ref = refs[-1]
    conf = jnp.sum(refs[0][...], axis=0, keepdims=True)
    for r in refs[1:-1]:
        conf = conf + jnp.sum(r[...], axis=0, keepdims=True)
    _iou_from_conf(conf, o_ref)


def _stage3(parts):
    return pl.pallas_call(
        _iou_multi_body,
        out_shape=jax.ShapeDtypeStruct((1, 1), jnp.float32),
        out_specs=pl.BlockSpec(memory_space=pltpu.SMEM),
    )(*parts)


SPLITS = (4, 4)


def kernel(preds, gts):
    gi = gts.astype(jnp.int32)
    parts = []
    b0 = 0
    for nb in SPLITS:
        idx = _stage1(preds, gi, b0, nb)
        parts.append(_stage2(idx))
        b0 += nb
    res = _stage3(parts)
    return res[0, 0]


# speedup vs baseline: 1.0418x; 1.0418x over previous
import jax
import jax.numpy as jnp
from jax import lax
from jax.experimental import pallas as pl
from jax.experimental.pallas import tpu as pltpu
from jax.experimental.pallas import tpu_sc as plsc

N = 21
B, H, W = 8, 512, 512
BH = 256
NBINS = N * N
PADB = 512
NC, NS = 2, 16
NW = NC * NS
VLEN = 16
TOTAL = B * H * W
CHUNK = TOTAL // NW


def _argmax_body(p_ref, g_ref, o_ref):
    x = p_ref[0]
    m = jnp.max(x, axis=0)
    ch = lax.broadcasted_iota(jnp.int32, (N, BH, W), 0)
    am = jnp.min(jnp.where(x == m[None], ch, N), axis=0)
    o_ref[...] = g_ref[0] * N + am


def _hist_body(idx_hbm, out_hbm, idx_v, hist_v, acc_v):
    rows = idx_hbm.shape[0] // NW
    wid = lax.axis_index("c") * NS + lax.axis_index("s")

    @plsc.parallel_loop(0, VLEN * PADB, step=VLEN, unroll=8)
    def _zero(i):
        hist_v[pl.ds(i, VLEN)] = jnp.zeros((VLEN,), jnp.float32)

    pltpu.sync_copy(idx_hbm.at[pl.ds(wid * rows, rows), :], idx_v)

    lanebase = lax.iota(jnp.int32, VLEN) * PADB
    ones = jnp.ones((VLEN,), jnp.float32)

    @plsc.parallel_loop(0, rows, step=1, unroll=4)
    def _scat(i):
        for c in range(W // VLEN):
            v = idx_v[i, pl.ds(c * VLEN, VLEN)]
            plsc.addupdate_scatter(hist_v, [lanebase + v], ones)

    @plsc.parallel_loop(0, PADB, step=VLEN, unroll=2)
    def _red(cc):
        acc = hist_v[pl.ds(cc, VLEN)]
        for l in range(1, VLEN):
            acc = acc + hist_v[pl.ds(l * PADB + cc, VLEN)]
        acc_v[pl.ds(cc, VLEN)] = acc

    pltpu.sync_copy(acc_v, out_hbm.at[wid])


def _iou_from_conf(conf, o_ref):
    confb = jnp.broadcast_to(conf, (N, PADB))
    b = lax.broadcasted_iota(jnp.int32, (N, PADB), 1)
    c = lax.broadcasted_iota(jnp.int32, (N, PADB), 0)
    valid = b < NBINS
    rowm = ((b // N) == c) & valid
    colm = (lax.rem(b, N) == c) & valid
    tpm = rowm & colm
    z = jnp.zeros((N, PADB), jnp.float32)
    rowsum = jnp.sum(jnp.where(rowm, confb, z), axis=1, keepdims=True)
    colsum = jnp.sum(jnp.where(colm, confb, z), axis=1, keepdims=True)
    tp = jnp.sum(jnp.where(tpm, confb, z), axis=1, keepdims=True)
    denom = rowsum + colsum - tp
    present = denom > 0.0
    iou = jnp.where(present, tp / jnp.where(present, denom, 1.0), 0.0)
    miou = jnp.sum(iou) / jnp.sum(present.astype(jnp.float32))
    o_ref[0, 0] = miou


def _stage1(preds, gts, b0, nb):
    return pl.pallas_call(
        _argmax_body,
        grid=(nb, H // BH),
        in_specs=[
            pl.BlockSpec((1, N, BH, W), lambda b, h: (b + b0, 0, h, 0)),
            pl.BlockSpec((1, BH, W), lambda b, h: (b + b0, h, 0)),
        ],
        out_specs=pl.BlockSpec((BH, W), lambda b, h: (b * (H // BH) + h, 0)),
        out_shape=jax.ShapeDtypeStruct((nb * H, W), jnp.int32),
    )(preds, gts)


def _stage2(idx):
    rows = idx.shape[0] // NW
    mesh = plsc.VectorSubcoreMesh(core_axis_name="c", subcore_axis_name="s")
    return pl.kernel(
        _hist_body,
        out_type=jax.ShapeDtypeStruct((NW, PADB), jnp.float32),
        mesh=mesh,
        scratch_types=[
            pltpu.VMEM((rows, W), jnp.int32),
            pltpu.VMEM((VLEN * PADB,), jnp.float32),
            pltpu.VMEM((PADB,), jnp.float32),
        ],
        compiler_params=pltpu.CompilerParams(
            needs_layout_passes=False, use_tc_tiling_on_sc=True),
    )(idx)


def _iou_multi_body(*refs):
    o_ref = refs[-1]
    conf = jnp.sum(refs[0][...], axis=0, keepdims=True)
    for r in refs[1:-1]:
        conf = conf + jnp.sum(r[...], axis=0, keepdims=True)
    _iou_from_conf(conf, o_ref)


def _stage3(parts):
    return pl.pallas_call(
        _iou_multi_body,
        out_shape=jax.ShapeDtypeStruct((1, 1), jnp.float32),
        out_specs=pl.BlockSpec(memory_space=pltpu.SMEM),
    )(*parts)


SPLITS = (5, 3)


def kernel(preds, gts):
    gi = gts.astype(jnp.int32)
    parts = []
    b0 = 0
    for nb in SPLITS:
        idx = _stage1(preds, gi, b0, nb)
        parts.append(_stage2(idx))
        b0 += nb
    res = _stage3(parts)
    return res[0, 0]


# speedup vs baseline: 1.0560x; 1.0136x over previous
import jax
import jax.numpy as jnp
from jax import lax
from jax.experimental import pallas as pl
from jax.experimental.pallas import tpu as pltpu
from jax.experimental.pallas import tpu_sc as plsc

N = 21
B, H, W = 8, 512, 512
BH = 256
NBINS = N * N
PADB = 512
NC, NS = 2, 16
NW = NC * NS
VLEN = 16
TOTAL = B * H * W
CHUNK = TOTAL // NW


def _argmax_body(p_ref, g_ref, o_ref):
    x = p_ref[0]
    m = jnp.max(x, axis=0)
    ch = lax.broadcasted_iota(jnp.int32, (N, BH, W), 0)
    am = jnp.min(jnp.where(x == m[None], ch, N), axis=0)
    o_ref[...] = g_ref[0] * N + am


def _hist_body(idx_hbm, out_hbm, idx_v, hist_v, acc_v):
    rows = idx_hbm.shape[0] // NW
    wid = lax.axis_index("c") * NS + lax.axis_index("s")

    @plsc.parallel_loop(0, VLEN * PADB, step=VLEN, unroll=8)
    def _zero(i):
        hist_v[pl.ds(i, VLEN)] = jnp.zeros((VLEN,), jnp.float32)

    pltpu.sync_copy(idx_hbm.at[pl.ds(wid * rows, rows), :], idx_v)

    lanebase = lax.iota(jnp.int32, VLEN) * PADB
    ones = jnp.ones((VLEN,), jnp.float32)

    @plsc.parallel_loop(0, rows, step=1, unroll=4)
    def _scat(i):
        for c in range(W // VLEN):
            v = idx_v[i, pl.ds(c * VLEN, VLEN)]
            plsc.addupdate_scatter(hist_v, [lanebase + v], ones)

    @plsc.parallel_loop(0, PADB, step=VLEN, unroll=2)
    def _red(cc):
        acc = hist_v[pl.ds(cc, VLEN)]
        for l in range(1, VLEN):
            acc = acc + hist_v[pl.ds(l * PADB + cc, VLEN)]
        acc_v[pl.ds(cc, VLEN)] = acc

    pltpu.sync_copy(acc_v, out_hbm.at[wid])


def _iou_from_conf(conf, o_ref):
    confb = jnp.broadcast_to(conf, (N, PADB))
    b = lax.broadcasted_iota(jnp.int32, (N, PADB), 1)
    c = lax.broadcasted_iota(jnp.int32, (N, PADB), 0)
    valid = b < NBINS
    rowm = ((b // N) == c) & valid
    colm = (lax.rem(b, N) == c) & valid
    tpm = rowm & colm
    z = jnp.zeros((N, PADB), jnp.float32)
    rowsum = jnp.sum(jnp.where(rowm, confb, z), axis=1, keepdims=True)
    colsum = jnp.sum(jnp.where(colm, confb, z), axis=1, keepdims=True)
    tp = jnp.sum(jnp.where(tpm, confb, z), axis=1, keepdims=True)
    denom = rowsum + colsum - tp
    present = denom > 0.0
    iou = jnp.where(present, tp / jnp.where(present, denom, 1.0), 0.0)
    miou = jnp.sum(iou) / jnp.sum(present.astype(jnp.float32))
    o_ref[0, 0] = miou


def _stage1(preds, gts, b0, nb):
    return pl.pallas_call(
        _argmax_body,
        grid=(nb, H // BH),
        in_specs=[
            pl.BlockSpec((1, N, BH, W), lambda b, h: (b + b0, 0, h, 0)),
            pl.BlockSpec((1, BH, W), lambda b, h: (b + b0, h, 0)),
        ],
        out_specs=pl.BlockSpec((BH, W), lambda b, h: (b * (H // BH) + h, 0)),
        out_shape=jax.ShapeDtypeStruct((nb * H, W), jnp.int32),
    )(preds, gts)


def _stage2(idx):
    rows = idx.shape[0] // NW
    mesh = plsc.VectorSubcoreMesh(core_axis_name="c", subcore_axis_name="s")
    return pl.kernel(
        _hist_body,
        out_type=jax.ShapeDtypeStruct((NW, PADB), jnp.float32),
        mesh=mesh,
        scratch_types=[
            pltpu.VMEM((rows, W), jnp.int32),
            pltpu.VMEM((VLEN * PADB,), jnp.float32),
            pltpu.VMEM((PADB,), jnp.float32),
        ],
        compiler_params=pltpu.CompilerParams(
            needs_layout_passes=False, use_tc_tiling_on_sc=True),
    )(idx)


def _iou_multi_body(*refs):
    o_ref = refs[-1]
    conf = jnp.sum(refs[0][...], axis=0, keepdims=True)
    for r in refs[1:-1]:
        conf = conf + jnp.sum(r[...], axis=0, keepdims=True)
    _iou_from_conf(conf, o_ref)


def _stage3(parts):
    return pl.pallas_call(
        _iou_multi_body,
        out_shape=jax.ShapeDtypeStruct((1, 1), jnp.float32),
        out_specs=pl.BlockSpec(memory_space=pltpu.SMEM),
    )(*parts)


SPLITS = (6, 2)


def kernel(preds, gts):
    gi = gts.astype(jnp.int32)
    parts = []
    b0 = 0
    for nb in SPLITS:
        idx = _stage1(preds, gi, b0, nb)
        parts.append(_stage2(idx))
        b0 += nb
    res = _stage3(parts)
    return res[0, 0]
